# baseline (speedup 1.0000x reference)
import functools

import jax
import jax.numpy as jnp
from jax import lax
from jax.experimental import pallas as pl
from jax.experimental.pallas import tpu as pltpu
from jax.experimental.pallas import tpu_sc as plsc

N = 10000
E = 320000
D = 128
H = 40
HP = 48
G = 64

NW = 32
EPW = E // NW
CHUNK = 128
NCH = 80
GN = 5
NGRP = NCH // (2 * GN)
EPW_PAD = NCH * CHUNK
NP = 10240
RPT = NP // 16
RPW = NP // 32


def _sc_scatter_body(y_hbm, srcw, dstw, zeros_hbm, out_hbm,
                     src_v, dst_v, rows_a, rows_b, acc,
                     sem_s, sem_ga, sem_gb, sem_sa, sem_sb):
    c = lax.axis_index("c")
    s = lax.axis_index("s")
    w = s * 2 + c
    rbase = s * RPT
    pltpu.async_copy(srcw.at[w], src_v, sem_s)
    pltpu.async_copy(dstw.at[w], dst_v, sem_s)
    pltpu.async_copy(zeros_hbm.at[pl.ds(rbase, RPT)], acc.at[pl.ds(rbase, RPT)],
                     sem_s)
    pltpu.make_async_copy(srcw.at[w], src_v, sem_s).wait()
    pltpu.make_async_copy(dstw.at[w], dst_v, sem_s).wait()
    pltpu.make_async_copy(zeros_hbm.at[pl.ds(rbase, RPT)],
                          acc.at[pl.ds(rbase, RPT)], sem_s).wait()
    plsc.subcore_barrier()

    def g_start(j, buf, sem):
        pltpu.async_copy(y_hbm.at[src_v.at[j]], buf, sem)

    def g_wait(j, buf, sem):
        pltpu.make_async_copy(y_hbm.at[src_v.at[j]], buf, sem).wait()

    def s_start(j, buf, sem):
        pltpu.async_copy(buf, acc.at[dst_v.at[j]], sem, add=True)

    def s_drain(j, buf, sem):
        pltpu.make_async_copy(buf, acc.at[dst_v.at[j]], sem).wait()

    for t in range(GN):
        g_start(t, rows_a.at[t], sem_ga)

    def body(k, carry):
        base = 2 * GN * k
        for t in range(GN):
            g_wait(base + t, rows_a.at[t], sem_ga)
            s_start(base + t, rows_a.at[t], sem_sa)
        for t in range(GN):
            g_start(base + GN + t, rows_b.at[t], sem_gb)
        for t in range(GN):
            g_wait(base + GN + t, rows_b.at[t], sem_gb)
            s_start(base + GN + t, rows_b.at[t], sem_sb)
        for t in range(GN):
            s_drain(base + t, rows_a.at[t], sem_sa)

        @pl.when(k + 1 < NGRP)
        def _():
            for t in range(GN):
                g_start(base + 2 * GN + t, rows_a.at[t], sem_ga)

        for t in range(GN):
            s_drain(base + GN + t, rows_b.at[t], sem_sb)
        return carry

    lax.fori_loop(0, NGRP, body, 0)
    plsc.subcore_barrier()
    pltpu.sync_copy(acc.at[pl.ds(rbase, RPT)], out_hbm.at[c, pl.ds(rbase, RPT)])


_sc_scatter = functools.partial(
    pl.kernel,
    mesh=plsc.VectorSubcoreMesh(core_axis_name="c", subcore_axis_name="s"),
    out_type=jax.ShapeDtypeStruct((2, NP, HP), jnp.float32),
    scratch_types=[
        pltpu.VMEM((NCH, CHUNK), jnp.int32),
        pltpu.VMEM((NCH, CHUNK), jnp.int32),
        pltpu.VMEM((GN, CHUNK, HP), jnp.float32),
        pltpu.VMEM((GN, CHUNK, HP), jnp.float32),
        pltpu.VMEM_SHARED((NP, HP), jnp.float32),
        pltpu.SemaphoreType.DMA,
        pltpu.SemaphoreType.DMA,
        pltpu.SemaphoreType.DMA,
        pltpu.SemaphoreType.DMA,
        pltpu.SemaphoreType.DMA,
    ],
    compiler_params=pltpu.CompilerParams(use_tc_tiling_on_sc=False),
)(_sc_scatter_body)


def _tc_pre_body(xp, wrel, wroot, y, r):
    xv = xp[...]
    y[...] = jnp.dot(xv, wrel[...], preferred_element_type=jnp.float32)
    r[...] = jnp.dot(xv, wroot[...], preferred_element_type=jnp.float32)


def _tc_pre(xp, wrel, wroot):
    return pl.pallas_call(
        _tc_pre_body,
        out_shape=[jax.ShapeDtypeStruct((NP, HP), jnp.float32),
                   jax.ShapeDtypeStruct((NP, HP), jnp.float32)],
    )(xp, wrel, wroot)


def _bn_relu(parts, r, brel, g, b, relu):
    z = parts[0] + parts[1] + r + brel
    if relu:
        z = jnp.maximum(z, 0.0)
    zs = z[:N]
    mean = jnp.sum(zs, axis=0, keepdims=True) / N
    zc = zs - mean
    var = jnp.sum(zc * zc, axis=0, keepdims=True) / N
    inv = g / jnp.sqrt(var + 1e-5)
    return (z - mean) * inv + b


def _tc_mid_body(parts, r, brel, g, b, wrel, wroot, y2, r2):
    h = _bn_relu(parts[...], r[...], brel[...], g[...], b[...], relu=True)
    y2[...] = jnp.dot(h, wrel[...], preferred_element_type=jnp.float32)
    r2[...] = jnp.dot(h, wroot[...], preferred_element_type=jnp.float32)


def _tc_mid(parts, r, brel, g, b, wrel, wroot):
    return pl.pallas_call(
        _tc_mid_body,
        out_shape=[jax.ShapeDtypeStruct((NP, HP), jnp.float32),
                   jax.ShapeDtypeStruct((NP, HP), jnp.float32)],
    )(parts, r, brel, g, b, wrel, wroot)


def _tc_zstats_body(parts, r, brel, z_out, stats_out):
    z = parts[0] + parts[1] + r[...] + brel[...]
    zs = z[:N]
    mean = jnp.sum(zs, axis=0, keepdims=True) / N
    zc = zs - mean
    var = jnp.sum(zc * zc, axis=0, keepdims=True) / N
    inv = 1.0 / jnp.sqrt(var + 1e-5)
    rowmask = lax.broadcasted_iota(jnp.int32, (NP, 1), 0) < N
    z_out[...] = jnp.where(rowmask, z, -jnp.inf)
    stats_out[...] = jnp.concatenate([mean, inv], axis=0)


def _tc_zstats(parts, r, brel):
    return pl.pallas_call(
        _tc_zstats_body,
        out_shape=[jax.ShapeDtypeStruct((NP, HP), jnp.float32),
                   jax.ShapeDtypeStruct((2, HP), jnp.float32)],
    )(parts, r, brel)


def _sc_pool_body(z_hbm, bids_hbm, out_hbm, z_v, bid_v, pooled_v, sem):
    c = lax.axis_index("c")
    s = lax.axis_index("s")
    w = s * 2 + c
    rbase = w * RPW
    pltpu.async_copy(z_hbm.at[pl.ds(rbase, RPW)], z_v, sem)
    pltpu.async_copy(bids_hbm.at[pl.ds(rbase, RPW)], bid_v.at[pl.ds(0, RPW)], sem)
    pltpu.make_async_copy(z_hbm.at[pl.ds(rbase, RPW)], z_v, sem).wait()
    pltpu.make_async_copy(bids_hbm.at[pl.ds(rbase, RPW)],
                          bid_v.at[pl.ds(0, RPW)], sem).wait()

    neg = jnp.full((16,), -jnp.inf, jnp.float32)

    def init_body(i, carry):
        pooled_v[pl.ds(i * 16, 16)] = neg
        return carry

    lax.fori_loop(0, G * HP // 16, init_body, 0)

    def row_body(i, carry):
        bid = bid_v[pl.ds(i, 16)][0]
        for kk in range(HP // 16):
            val = z_v[i, pl.ds(kk * 16, 16)]
            cur = pooled_v[pl.ds(bid * HP + kk * 16, 16)]
            pooled_v[pl.ds(bid * HP + kk * 16, 16)] = jnp.maximum(cur, val)
        return carry

    lax.fori_loop(0, RPW, row_body, 0)
    pltpu.sync_copy(pooled_v, out_hbm.at[w])


_sc_pool = functools.partial(
    pl.kernel,
    mesh=plsc.VectorSubcoreMesh(core_axis_name="c", subcore_axis_name="s"),
    out_type=jax.ShapeDtypeStruct((NW, G * HP), jnp.float32),
    scratch_types=[
        pltpu.VMEM((RPW, HP), jnp.float32),
        pltpu.VMEM((RPW + 16,), jnp.int32),
        pltpu.VMEM((G * HP,), jnp.float32),
        pltpu.SemaphoreType.DMA,
    ],
    compiler_params=pltpu.CompilerParams(use_tc_tiling_on_sc=False),
)(_sc_pool_body)


def _tc_head_body(pool_parts, stats, g, b, wl1, bl1, wl2, bl2, out):
    pooled = jnp.max(pool_parts[...], axis=0)
    mean = stats[0:1, :]
    inv = stats[1:2, :]
    pb = (pooled - mean) * (inv * g[...]) + b[...]
    t = jnp.dot(pb, wl1[...], preferred_element_type=jnp.float32) + bl1[...]
    t = jnp.maximum(t, 0.0)
    t = jnp.dot(t, wl2[...], preferred_element_type=jnp.float32) + bl2[...]
    out[...] = 1.0 / (1.0 + jnp.exp(-t[:, 0:1]))


def _tc_head(pool_parts, stats, g, b, wl1, bl1, wl2, bl2):
    return pl.pallas_call(
        _tc_head_body,
        out_shape=jax.ShapeDtypeStruct((G, 1), jnp.float32),
    )(pool_parts, stats, g, b, wl1, bl1, wl2, bl2)


def _pad_w(w, rows, cols):
    out = jnp.zeros((rows, cols), jnp.float32)
    return out.at[:w.shape[0], :w.shape[1]].set(w)


def _pad_v(v, cols):
    return jnp.zeros((1, cols), jnp.float32).at[0, :v.shape[0]].set(v)


@jax.jit
def kernel(x, edge_index, edge_attr, batch,
           Wrel1, brel1, Wroot1, Wrel2, brel2, Wroot2, Wrel3, brel3, Wroot3,
           g1, b1, g2, b2, g3, b3, Wl1, bl1, Wl2, bl2):
    xp = jnp.zeros((NP, D), jnp.float32).at[:N].set(x)
    src = jnp.full((EPW_PAD * NW,), N, jnp.int32).at[:E].set(edge_index[0])
    dst = jnp.full((EPW_PAD * NW,), N, jnp.int32).at[:E].set(edge_index[1])
    srcw = src.reshape(NW, NCH, CHUNK)
    dstw = dst.reshape(NW, NCH, CHUNK)
    zeros_hbm = jnp.zeros((NP, HP), jnp.float32)
    bids = jnp.full((NP,), G - 1, jnp.int32).at[:N].set(batch)

    wrel1 = _pad_w(Wrel1, D, HP)
    wroot1 = _pad_w(Wroot1, D, HP)
    wrel2 = _pad_w(Wrel2, HP, HP)
    wroot2 = _pad_w(Wroot2, HP, HP)
    wrel3 = _pad_w(Wrel3, HP, HP)
    wroot3 = _pad_w(Wroot3, HP, HP)
    wl1 = _pad_w(Wl1, HP, 128)
    wl2 = _pad_w(Wl2, 128, 128)
    pb1, pg1, pbt1 = _pad_v(brel1, HP), _pad_v(g1, HP), _pad_v(b1, HP)
    pb2, pg2, pbt2 = _pad_v(brel2, HP), _pad_v(g2, HP), _pad_v(b2, HP)
    pb3, pg3, pbt3 = _pad_v(brel3, HP), _pad_v(g3, HP), _pad_v(b3, HP)
    pbl1 = _pad_v(bl1, 128)
    pbl2 = _pad_v(bl2, 128)

    y1, r1 = _tc_pre(xp, wrel1, wroot1)
    parts1 = _sc_scatter(y1, srcw, dstw, zeros_hbm)
    y2, r2 = _tc_mid(parts1, r1, pb1, pg1, pbt1, wrel2, wroot2)
    parts2 = _sc_scatter(y2, srcw, dstw, zeros_hbm)
    y3, r3 = _tc_mid(parts2, r2, pb2, pg2, pbt2, wrel3, wroot3)
    parts3 = _sc_scatter(y3, srcw, dstw, zeros_hbm)
    z3, stats3 = _tc_zstats(parts3, r3, pb3)
    pool_parts = _sc_pool(z3, bids).reshape(NW, G, HP)
    return _tc_head(pool_parts, stats3, pg3, pbt3, wl1, pbl1, wl2, pbl2)


# speedup vs baseline: 1.9447x; 1.9447x over previous
import functools

import jax
import jax.numpy as jnp
from jax import lax
from jax.experimental import pallas as pl
from jax.experimental.pallas import tpu as pltpu
from jax.experimental.pallas import tpu_sc as plsc

N = 10000
E = 320000
D = 128
H = 40
HP = 48
G = 64

NW = 32
EPW = E // NW
CHUNK = 128
NCH = 80
GN = 2
NGRP = NCH // (2 * GN)
EPW_PAD = NCH * CHUNK
NP = 10240
RPT = NP // 16
RPW = NP // 32


def _sc_scatter_body(y_hbm, srcw, dstw, zeros_hbm, out_hbm,
                     src_v, dst_v, rows_a, rows_b, y_spm, acc,
                     sem_s, sem_ga, sem_gb, sem_sa, sem_sb):
    c = lax.axis_index("c")
    s = lax.axis_index("s")
    w = s * 2 + c
    rbase = s * RPT
    pltpu.async_copy(srcw.at[w], src_v, sem_s)
    pltpu.async_copy(dstw.at[w], dst_v, sem_s)
    pltpu.async_copy(y_hbm.at[pl.ds(rbase, RPT)], y_spm.at[pl.ds(rbase, RPT)],
                     sem_s)
    pltpu.async_copy(zeros_hbm.at[pl.ds(rbase, RPT)], acc.at[pl.ds(rbase, RPT)],
                     sem_s)
    pltpu.make_async_copy(srcw.at[w], src_v, sem_s).wait()
    pltpu.make_async_copy(dstw.at[w], dst_v, sem_s).wait()
    pltpu.make_async_copy(y_hbm.at[pl.ds(rbase, RPT)],
                          y_spm.at[pl.ds(rbase, RPT)], sem_s).wait()
    pltpu.make_async_copy(zeros_hbm.at[pl.ds(rbase, RPT)],
                          acc.at[pl.ds(rbase, RPT)], sem_s).wait()
    plsc.subcore_barrier()

    def g_start(j, buf, sem):
        pltpu.async_copy(y_spm.at[src_v.at[j]], buf, sem)

    def g_wait(j, buf, sem):
        pltpu.make_async_copy(y_spm.at[src_v.at[j]], buf, sem).wait()

    def s_start(j, buf, sem):
        pltpu.async_copy(buf, acc.at[dst_v.at[j]], sem, add=True)

    def s_drain(j, buf, sem):
        pltpu.make_async_copy(buf, acc.at[dst_v.at[j]], sem).wait()

    for t in range(GN):
        g_start(t, rows_a.at[t], sem_ga)

    def body(k, carry):
        base = 2 * GN * k
        for t in range(GN):
            g_wait(base + t, rows_a.at[t], sem_ga)
            s_start(base + t, rows_a.at[t], sem_sa)
        for t in range(GN):
            g_start(base + GN + t, rows_b.at[t], sem_gb)
        for t in range(GN):
            g_wait(base + GN + t, rows_b.at[t], sem_gb)
            s_start(base + GN + t, rows_b.at[t], sem_sb)
        for t in range(GN):
            s_drain(base + t, rows_a.at[t], sem_sa)

        @pl.when(k + 1 < NGRP)
        def _():
            for t in range(GN):
                g_start(base + 2 * GN + t, rows_a.at[t], sem_ga)

        for t in range(GN):
            s_drain(base + GN + t, rows_b.at[t], sem_sb)
        return carry

    lax.fori_loop(0, NGRP, body, 0)
    plsc.subcore_barrier()
    pltpu.sync_copy(acc.at[pl.ds(rbase, RPT)], out_hbm.at[c, pl.ds(rbase, RPT)])


_sc_scatter = functools.partial(
    pl.kernel,
    mesh=plsc.VectorSubcoreMesh(core_axis_name="c", subcore_axis_name="s"),
    out_type=jax.ShapeDtypeStruct((2, NP, HP), jnp.float32),
    scratch_types=[
        pltpu.VMEM((NCH, CHUNK), jnp.int32),
        pltpu.VMEM((NCH, CHUNK), jnp.int32),
        pltpu.VMEM((GN, CHUNK, HP), jnp.float32),
        pltpu.VMEM((GN, CHUNK, HP), jnp.float32),
        pltpu.VMEM_SHARED((NP, HP), jnp.float32),
        pltpu.VMEM_SHARED((NP, HP), jnp.float32),
        pltpu.SemaphoreType.DMA,
        pltpu.SemaphoreType.DMA,
        pltpu.SemaphoreType.DMA,
        pltpu.SemaphoreType.DMA,
        pltpu.SemaphoreType.DMA,
    ],
    compiler_params=pltpu.CompilerParams(use_tc_tiling_on_sc=False),
)(_sc_scatter_body)


def _tc_pre_body(xp, wrel, wroot, y, r):
    xv = xp[...]
    y[...] = jnp.dot(xv, wrel[...], preferred_element_type=jnp.float32)
    r[...] = jnp.dot(xv, wroot[...], preferred_element_type=jnp.float32)


def _tc_pre(xp, wrel, wroot):
    return pl.pallas_call(
        _tc_pre_body,
        out_shape=[jax.ShapeDtypeStruct((NP, HP), jnp.float32),
                   jax.ShapeDtypeStruct((NP, HP), jnp.float32)],
    )(xp, wrel, wroot)


def _bn_relu(parts, r, brel, g, b, relu):
    z = parts[0] + parts[1] + r + brel
    if relu:
        z = jnp.maximum(z, 0.0)
    zs = z[:N]
    mean = jnp.sum(zs, axis=0, keepdims=True) / N
    zc = zs - mean
    var = jnp.sum(zc * zc, axis=0, keepdims=True) / N
    inv = g / jnp.sqrt(var + 1e-5)
    return (z - mean) * inv + b


def _tc_mid_body(parts, r, brel, g, b, wrel, wroot, y2, r2):
    h = _bn_relu(parts[...], r[...], brel[...], g[...], b[...], relu=True)
    y2[...] = jnp.dot(h, wrel[...], preferred_element_type=jnp.float32)
    r2[...] = jnp.dot(h, wroot[...], preferred_element_type=jnp.float32)


def _tc_mid(parts, r, brel, g, b, wrel, wroot):
    return pl.pallas_call(
        _tc_mid_body,
        out_shape=[jax.ShapeDtypeStruct((NP, HP), jnp.float32),
                   jax.ShapeDtypeStruct((NP, HP), jnp.float32)],
    )(parts, r, brel, g, b, wrel, wroot)


def _tc_final_body(parts, r, brel, g, b, batch2d, wl1, bl1, wl2, bl2, out):
    h = _bn_relu(parts[...], r[...], brel[...], g[...], b[...], relu=False)
    hs = h[:N]
    bvec = batch2d[...]
    neg = jnp.float32(-jnp.inf)
    gids = lax.broadcasted_iota(jnp.int32, (G, 1), 0)

    def pool_body(gid, pooled):
        val = jnp.max(jnp.where(bvec == gid, hs, neg), axis=0, keepdims=True)
        return jnp.where(gids == gid, val, pooled)

    pooled = lax.fori_loop(0, G, pool_body, jnp.full((G, HP), neg))
    t = jnp.dot(pooled, wl1[...], preferred_element_type=jnp.float32) + bl1[...]
    t = jnp.maximum(t, 0.0)
    t = jnp.dot(t, wl2[...], preferred_element_type=jnp.float32) + bl2[...]
    out[...] = 1.0 / (1.0 + jnp.exp(-t[:, 0:1]))


def _tc_final(parts, r, brel, g, b, batch2d, wl1, bl1, wl2, bl2):
    return pl.pallas_call(
        _tc_final_body,
        out_shape=jax.ShapeDtypeStruct((G, 1), jnp.float32),
    )(parts, r, brel, g, b, batch2d, wl1, bl1, wl2, bl2)


def _pad_w(w, rows, cols):
    out = jnp.zeros((rows, cols), jnp.float32)
    return out.at[:w.shape[0], :w.shape[1]].set(w)


def _pad_v(v, cols):
    return jnp.zeros((1, cols), jnp.float32).at[0, :v.shape[0]].set(v)


@jax.jit
def kernel(x, edge_index, edge_attr, batch,
           Wrel1, brel1, Wroot1, Wrel2, brel2, Wroot2, Wrel3, brel3, Wroot3,
           g1, b1, g2, b2, g3, b3, Wl1, bl1, Wl2, bl2):
    xp = jnp.zeros((NP, D), jnp.float32).at[:N].set(x)
    src = jnp.full((EPW_PAD * NW,), N, jnp.int32).at[:E].set(edge_index[0])
    dst = jnp.full((EPW_PAD * NW,), N, jnp.int32).at[:E].set(edge_index[1])
    srcw = src.reshape(NW, NCH, CHUNK)
    dstw = dst.reshape(NW, NCH, CHUNK)
    zeros_hbm = jnp.zeros((NP, HP), jnp.float32)
    batch2d = batch.reshape(N, 1)

    wrel1 = _pad_w(Wrel1, D, HP)
    wroot1 = _pad_w(Wroot1, D, HP)
    wrel2 = _pad_w(Wrel2, HP, HP)
    wroot2 = _pad_w(Wroot2, HP, HP)
    wrel3 = _pad_w(Wrel3, HP, HP)
    wroot3 = _pad_w(Wroot3, HP, HP)
    wl1 = _pad_w(Wl1, HP, 128)
    wl2 = _pad_w(Wl2, 128, 128)
    pb1, pg1, pbt1 = _pad_v(brel1, HP), _pad_v(g1, HP), _pad_v(b1, HP)
    pb2, pg2, pbt2 = _pad_v(brel2, HP), _pad_v(g2, HP), _pad_v(b2, HP)
    pb3, pg3, pbt3 = _pad_v(brel3, HP), _pad_v(g3, HP), _pad_v(b3, HP)
    pbl1 = _pad_v(bl1, 128)
    pbl2 = _pad_v(bl2, 128)

    y1, r1 = _tc_pre(xp, wrel1, wroot1)
    parts1 = _sc_scatter(y1, srcw, dstw, zeros_hbm)
    y2, r2 = _tc_mid(parts1, r1, pb1, pg1, pbt1, wrel2, wroot2)
    parts2 = _sc_scatter(y2, srcw, dstw, zeros_hbm)
    y3, r3 = _tc_mid(parts2, r2, pb2, pg2, pbt2, wrel3, wroot3)
    parts3 = _sc_scatter(y3, srcw, dstw, zeros_hbm)
    return _tc_final(parts3, r3, pb3, pg3, pbt3, batch2d, wl1, pbl1, wl2, pbl2)


# speedup vs baseline: 2.0082x; 1.0327x over previous
import functools

import jax
import jax.numpy as jnp
from jax import lax
from jax.experimental import pallas as pl
from jax.experimental.pallas import tpu as pltpu
from jax.experimental.pallas import tpu_sc as plsc

N = 10000
E = 320000
D = 128
H = 40
HP = 48
G = 64

NW = 32
EPW = E // NW
CHUNK = 128
NCH = 80
GN = 3
NGRP = NCH // (2 * GN)
EPW_PAD = NCH * CHUNK
NP = 10240
RPT = NP // 16
RPW = NP // 32


def _sc_scatter_body(y_hbm, srcw, dstw, zeros_hbm, out_hbm,
                     src_v, dst_v, rows_a, rows_b, y_spm, acc,
                     sem_s, sem_ga, sem_gb, sem_sa, sem_sb):
    c = lax.axis_index("c")
    s = lax.axis_index("s")
    w = s * 2 + c
    rbase = s * RPT
    pltpu.async_copy(srcw.at[w], src_v, sem_s)
    pltpu.async_copy(dstw.at[w], dst_v, sem_s)
    pltpu.async_copy(y_hbm.at[pl.ds(rbase, RPT)], y_spm.at[pl.ds(rbase, RPT)],
                     sem_s)
    pltpu.async_copy(zeros_hbm.at[pl.ds(rbase, RPT)], acc.at[pl.ds(rbase, RPT)],
                     sem_s)
    pltpu.make_async_copy(srcw.at[w], src_v, sem_s).wait()
    pltpu.make_async_copy(dstw.at[w], dst_v, sem_s).wait()
    pltpu.make_async_copy(y_hbm.at[pl.ds(rbase, RPT)],
                          y_spm.at[pl.ds(rbase, RPT)], sem_s).wait()
    pltpu.make_async_copy(zeros_hbm.at[pl.ds(rbase, RPT)],
                          acc.at[pl.ds(rbase, RPT)], sem_s).wait()
    plsc.subcore_barrier()

    def g_start(j, buf, sem):
        pltpu.async_copy(y_spm.at[src_v.at[j]], buf, sem)

    def g_wait(j, buf, sem):
        pltpu.make_async_copy(y_spm.at[src_v.at[j]], buf, sem).wait()

    def s_start(j, buf, sem):
        pltpu.async_copy(buf, acc.at[dst_v.at[j]], sem, add=True)

    def s_drain(j, buf, sem):
        pltpu.make_async_copy(buf, acc.at[dst_v.at[j]], sem).wait()

    for t in range(GN):
        g_start(t, rows_a.at[t], sem_ga)

    def body(k, carry):
        base = 2 * GN * k
        for t in range(GN):
            g_wait(base + t, rows_a.at[t], sem_ga)
            s_start(base + t, rows_a.at[t], sem_sa)
        for t in range(GN):
            g_start(base + GN + t, rows_b.at[t], sem_gb)
        for t in range(GN):
            g_wait(base + GN + t, rows_b.at[t], sem_gb)
            s_start(base + GN + t, rows_b.at[t], sem_sb)
        for t in range(GN):
            s_drain(base + t, rows_a.at[t], sem_sa)

        @pl.when(k + 1 < NGRP)
        def _():
            for t in range(GN):
                g_start(base + 2 * GN + t, rows_a.at[t], sem_ga)

        for t in range(GN):
            s_drain(base + GN + t, rows_b.at[t], sem_sb)
        return carry

    lax.fori_loop(0, NGRP, body, 0)
    plsc.subcore_barrier()
    pltpu.sync_copy(acc.at[pl.ds(rbase, RPT)], out_hbm.at[c, pl.ds(rbase, RPT)])


_sc_scatter = functools.partial(
    pl.kernel,
    mesh=plsc.VectorSubcoreMesh(core_axis_name="c", subcore_axis_name="s"),
    out_type=jax.ShapeDtypeStruct((2, NP, HP), jnp.float32),
    scratch_types=[
        pltpu.VMEM((NCH, CHUNK), jnp.int32),
        pltpu.VMEM((NCH, CHUNK), jnp.int32),
        pltpu.VMEM((GN, CHUNK, HP), jnp.float32),
        pltpu.VMEM((GN, CHUNK, HP), jnp.float32),
        pltpu.VMEM_SHARED((NP, HP), jnp.float32),
        pltpu.VMEM_SHARED((NP, HP), jnp.float32),
        pltpu.SemaphoreType.DMA,
        pltpu.SemaphoreType.DMA,
        pltpu.SemaphoreType.DMA,
        pltpu.SemaphoreType.DMA,
        pltpu.SemaphoreType.DMA,
    ],
    compiler_params=pltpu.CompilerParams(use_tc_tiling_on_sc=False),
)(_sc_scatter_body)


def _tc_pre_body(x, wrel, wroot, y, r):
    xv = x[...]
    zpad = jnp.zeros((NP - N, HP), jnp.float32)
    y[:N] = jnp.dot(xv, wrel[...], preferred_element_type=jnp.float32)
    y[N:] = zpad
    r[:N] = jnp.dot(xv, wroot[...], preferred_element_type=jnp.float32)
    r[N:] = zpad


def _tc_pre(x, wrel, wroot):
    return pl.pallas_call(
        _tc_pre_body,
        out_shape=[jax.ShapeDtypeStruct((NP, HP), jnp.float32),
                   jax.ShapeDtypeStruct((NP, HP), jnp.float32)],
    )(x, wrel, wroot)


def _bn_relu(parts, r, brel, g, b, relu):
    z = parts[0] + parts[1] + r + brel
    if relu:
        z = jnp.maximum(z, 0.0)
    zs = z[:N]
    mean = jnp.sum(zs, axis=0, keepdims=True) / N
    zc = zs - mean
    var = jnp.sum(zc * zc, axis=0, keepdims=True) / N
    inv = g / jnp.sqrt(var + 1e-5)
    return (z - mean) * inv + b


def _tc_mid_body(parts, r, brel, g, b, wrel, wroot, y2, r2):
    h = _bn_relu(parts[...], r[...], brel[...], g[...], b[...], relu=True)
    y2[...] = jnp.dot(h, wrel[...], preferred_element_type=jnp.float32)
    r2[...] = jnp.dot(h, wroot[...], preferred_element_type=jnp.float32)


def _tc_mid(parts, r, brel, g, b, wrel, wroot):
    return pl.pallas_call(
        _tc_mid_body,
        out_shape=[jax.ShapeDtypeStruct((NP, HP), jnp.float32),
                   jax.ShapeDtypeStruct((NP, HP), jnp.float32)],
    )(parts, r, brel, g, b, wrel, wroot)


def _tc_final_body(parts, r, brel, g, b, batch2d, wl1, bl1, wl2, bl2, out):
    h = _bn_relu(parts[...], r[...], brel[...], g[...], b[...], relu=False)
    hs = h[:N]
    bvec = batch2d[...]
    neg = jnp.float32(-jnp.inf)
    gids = lax.broadcasted_iota(jnp.int32, (G, 1), 0)

    def pool_body(gid, pooled):
        val = jnp.max(jnp.where(bvec == gid, hs, neg), axis=0, keepdims=True)
        return jnp.where(gids == gid, val, pooled)

    pooled = lax.fori_loop(0, G, pool_body, jnp.full((G, HP), neg))
    t = jnp.dot(pooled, wl1[...], preferred_element_type=jnp.float32) + bl1[...]
    t = jnp.maximum(t, 0.0)
    t = jnp.dot(t, wl2[...], preferred_element_type=jnp.float32) + bl2[...]
    out[...] = 1.0 / (1.0 + jnp.exp(-t[:, 0:1]))


def _tc_final(parts, r, brel, g, b, batch2d, wl1, bl1, wl2, bl2):
    return pl.pallas_call(
        _tc_final_body,
        out_shape=jax.ShapeDtypeStruct((G, 1), jnp.float32),
    )(parts, r, brel, g, b, batch2d, wl1, bl1, wl2, bl2)


def _pad_w(w, rows, cols):
    out = jnp.zeros((rows, cols), jnp.float32)
    return out.at[:w.shape[0], :w.shape[1]].set(w)


def _pad_v(v, cols):
    return jnp.zeros((1, cols), jnp.float32).at[0, :v.shape[0]].set(v)


@jax.jit
def kernel(x, edge_index, edge_attr, batch,
           Wrel1, brel1, Wroot1, Wrel2, brel2, Wroot2, Wrel3, brel3, Wroot3,
           g1, b1, g2, b2, g3, b3, Wl1, bl1, Wl2, bl2):
    src = jnp.full((EPW_PAD * NW,), N, jnp.int32).at[:E].set(edge_index[0])
    dst = jnp.full((EPW_PAD * NW,), N, jnp.int32).at[:E].set(edge_index[1])
    srcw = src.reshape(NW, NCH, CHUNK)
    dstw = dst.reshape(NW, NCH, CHUNK)
    zeros_hbm = jnp.zeros((NP, HP), jnp.float32)
    batch2d = batch.reshape(N, 1)

    wrel1 = _pad_w(Wrel1, D, HP)
    wroot1 = _pad_w(Wroot1, D, HP)
    wrel2 = _pad_w(Wrel2, HP, HP)
    wroot2 = _pad_w(Wroot2, HP, HP)
    wrel3 = _pad_w(Wrel3, HP, HP)
    wroot3 = _pad_w(Wroot3, HP, HP)
    wl1 = _pad_w(Wl1, HP, 128)
    wl2 = _pad_w(Wl2, 128, 128)
    pb1, pg1, pbt1 = _pad_v(brel1, HP), _pad_v(g1, HP), _pad_v(b1, HP)
    pb2, pg2, pbt2 = _pad_v(brel2, HP), _pad_v(g2, HP), _pad_v(b2, HP)
    pb3, pg3, pbt3 = _pad_v(brel3, HP), _pad_v(g3, HP), _pad_v(b3, HP)
    pbl1 = _pad_v(bl1, 128)
    pbl2 = _pad_v(bl2, 128)

    y1, r1 = _tc_pre(x, wrel1, wroot1)
    parts1 = _sc_scatter(y1, srcw, dstw, zeros_hbm)
    y2, r2 = _tc_mid(parts1, r1, pb1, pg1, pbt1, wrel2, wroot2)
    parts2 = _sc_scatter(y2, srcw, dstw, zeros_hbm)
    y3, r3 = _tc_mid(parts2, r2, pb2, pg2, pbt2, wrel3, wroot3)
    parts3 = _sc_scatter(y3, srcw, dstw, zeros_hbm)
    return _tc_final(parts3, r3, pb3, pg3, pbt3, batch2d, wl1, pbl1, wl2, pbl2)
